# baseline (speedup 1.0000x reference)
import jax
import jax.numpy as jnp
from jax import lax
from jax.experimental import pallas as pl
from jax.experimental.pallas import tpu as pltpu
from jax.experimental.pallas import tpu_sc as plsc

N = 320000
D = 128
S = 2048

NUM_CORES = 2
NUM_SUBCORES = 16
NUM_WORKERS = NUM_CORES * NUM_SUBCORES

CHUNK_ROWS = 256
IDX_ROWS = CHUNK_ROWS // 128
NUM_CHUNKS = N // CHUNK_ROWS
BASE_STEPS = NUM_CHUNKS // NUM_WORKERS
REMAINDER = NUM_CHUNKS - BASE_STEPS * NUM_WORKERS
MAX_STEPS = BASE_STEPS + 1
IDX_WIN = MAX_STEPS * IDX_ROWS + 8
IDX_PAD_ROWS = N // 128 + 4


def _sc_partials_body(nf_hbm, batch_hbm, part_hbm, rows_v, idx_all, acc, sems):
    c = lax.axis_index("c")
    s = lax.axis_index("s")
    w = s * NUM_CORES + c

    n = jnp.where(w < REMAINDER, BASE_STEPS + 1, BASE_STEPS)
    start = w * BASE_STEPS + jnp.minimum(w, REMAINDER)

    aligned = (start * IDX_ROWS) // 8 * 8
    delta = start * IDX_ROWS - aligned
    pltpu.async_copy(
        batch_hbm.at[pl.ds(aligned, IDX_WIN)],
        idx_all,
        sems.at[2],
    )

    def zero_row(i, _):
        for j in range(D // 16):
            rows_v[0, i, pl.ds(j * 16, 16)] = jnp.zeros((16,), jnp.float32)
        return 0

    lax.fori_loop(0, 128, zero_row, 0)
    pltpu.sync_copy(rows_v.at[0, pl.ds(0, 128)], acc.at[pl.ds(s * 128, 128)])

    def issue_rows(t, p):
        pltpu.async_copy(
            nf_hbm.at[pl.ds((start + t) * CHUNK_ROWS, CHUNK_ROWS)],
            rows_v.at[p],
            sems.at[p],
        )

    def wait_rows(p):
        pltpu.make_async_copy(
            nf_hbm.at[pl.ds(0, CHUNK_ROWS)], rows_v.at[p], sems.at[p]
        ).wait()

    def scatter(t, p):
        for j in range(IDX_ROWS):
            pltpu.sync_copy(
                rows_v.at[p, pl.ds(j * 128, 128)],
                acc.at[idx_all.at[delta + t * IDX_ROWS + j]],
                add=True,
            )

    issue_rows(0, 0)

    pltpu.make_async_copy(
        batch_hbm.at[pl.ds(0, IDX_WIN)],
        idx_all,
        sems.at[2],
    ).wait()

    plsc.subcore_barrier()

    def two_steps(i, _):
        t0 = 2 * i
        t1 = 2 * i + 1

        @pl.when(t0 < n)
        def _():
            wait_rows(0)

            @pl.when(t1 < n)
            def _():
                issue_rows(t1, 1)

            scatter(t0, 0)

        @pl.when(t1 < n)
        def _():
            wait_rows(1)

            @pl.when(t1 + 1 < n)
            def _():
                issue_rows(t1 + 1, 0)

            scatter(t1, 1)

        return 0

    lax.fori_loop(0, (MAX_STEPS + 1) // 2, two_steps, 0)

    plsc.subcore_barrier()
    pltpu.sync_copy(acc.at[pl.ds(s * 128, 128)], rows_v.at[0, pl.ds(0, 128)])
    pltpu.sync_copy(
        rows_v.at[0, pl.ds(0, 128)], part_hbm.at[c, pl.ds(s * 128, 128)]
    )


_sc_partials = pl.kernel(
    _sc_partials_body,
    out_type=jax.ShapeDtypeStruct((NUM_CORES, S, D), jnp.float32),
    mesh=plsc.VectorSubcoreMesh(core_axis_name="c", subcore_axis_name="s"),
    scratch_types=[
        pltpu.VMEM((2, CHUNK_ROWS, D), jnp.float32),
        pltpu.VMEM((IDX_WIN, 128), jnp.int32),
        pltpu.VMEM_SHARED((S, D), jnp.float32),
        pltpu.SemaphoreType.DMA((3,)),
    ],
)


def _combine_body(p_ref, o_ref):
    o_ref[...] = p_ref[0] + p_ref[1]


_combine = pl.pallas_call(
    _combine_body,
    out_shape=jax.ShapeDtypeStruct((S, D), jnp.float32),
)


def kernel(node_features, batch, ptr):
    del ptr
    batch2d = jnp.pad(
        batch.reshape(N // 128, 128), ((0, IDX_PAD_ROWS - N // 128), (0, 0))
    )
    partials = _sc_partials(node_features, batch2d)
    return _combine(partials)


# speedup vs baseline: 1.2864x; 1.2864x over previous
import jax
import jax.numpy as jnp
from jax import lax
from jax.experimental import pallas as pl
from jax.experimental.pallas import tpu as pltpu
from jax.experimental.pallas import tpu_sc as plsc

N = 320000
D = 128
S = 2048

NUM_CORES = 2
NUM_SUBCORES = 16
NUM_WORKERS = NUM_CORES * NUM_SUBCORES

CHUNK_ROWS = 256
IDX_ROWS = CHUNK_ROWS // 128
NUM_CHUNKS = N // CHUNK_ROWS
BASE_STEPS = NUM_CHUNKS // NUM_WORKERS
REMAINDER = NUM_CHUNKS - BASE_STEPS * NUM_WORKERS
MAX_STEPS = BASE_STEPS + (1 if REMAINDER else 0)
NBUF = 3


def _sc_partials_body(nf_hbm, batch_hbm, part_hbm, rows_v, idx_v, acc, ld_sem):
    c = lax.axis_index("c")
    s = lax.axis_index("s")
    w = s * NUM_CORES + c

    n = jnp.where(w < REMAINDER, BASE_STEPS + 1, BASE_STEPS)

    def chunk_of(t):
        return jnp.where(
            t < BASE_STEPS, w + NUM_WORKERS * t, BASE_STEPS * NUM_WORKERS + w
        )

    def issue_loads(t, p):
        chunk = chunk_of(t)
        pltpu.async_copy(
            nf_hbm.at[pl.ds(chunk * CHUNK_ROWS, CHUNK_ROWS)],
            rows_v.at[p],
            ld_sem.at[p],
        )
        pltpu.async_copy(
            batch_hbm.at[pl.ds(chunk * IDX_ROWS, IDX_ROWS)],
            idx_v.at[p],
            ld_sem.at[p],
        )

    def wait_loads(p):
        pltpu.make_async_copy(
            nf_hbm.at[pl.ds(0, CHUNK_ROWS)], rows_v.at[p], ld_sem.at[p]
        ).wait()
        pltpu.make_async_copy(
            batch_hbm.at[pl.ds(0, IDX_ROWS)], idx_v.at[p], ld_sem.at[p]
        ).wait()

    def scatter(p):
        for j in range(IDX_ROWS):
            pltpu.sync_copy(
                rows_v.at[p, pl.ds(j * 128, 128)], acc.at[idx_v.at[p, j]], add=True
            )

    issue_loads(0, 0)
    issue_loads(1, 1)

    def zero_row(i, _):
        for j in range(D // 16):
            rows_v[2, i, pl.ds(j * 16, 16)] = jnp.zeros((16,), jnp.float32)
        return 0

    lax.fori_loop(0, 128, zero_row, 0)
    pltpu.sync_copy(rows_v.at[2, pl.ds(0, 128)], acc.at[pl.ds(s * 128, 128)])
    plsc.subcore_barrier()

    def steps(i, _):
        for k in range(NBUF):
            t = NBUF * i + k

            @pl.when(t < n)
            def _():
                wait_loads(k)

                @pl.when(t + 2 < n)
                def _():
                    issue_loads(t + 2, (k + 2) % NBUF)

                scatter(k)

        return 0

    lax.fori_loop(0, (MAX_STEPS + NBUF - 1) // NBUF, steps, 0)

    plsc.subcore_barrier()
    pltpu.sync_copy(acc.at[pl.ds(s * 128, 128)], rows_v.at[0, pl.ds(0, 128)])
    pltpu.sync_copy(
        rows_v.at[0, pl.ds(0, 128)], part_hbm.at[c, pl.ds(s * 128, 128)]
    )


_sc_partials = pl.kernel(
    _sc_partials_body,
    out_type=jax.ShapeDtypeStruct((NUM_CORES, S, D), jnp.float32),
    mesh=plsc.VectorSubcoreMesh(core_axis_name="c", subcore_axis_name="s"),
    scratch_types=[
        pltpu.VMEM((NBUF, CHUNK_ROWS, D), jnp.float32),
        pltpu.VMEM((NBUF, IDX_ROWS, 128), jnp.int32),
        pltpu.VMEM_SHARED((S, D), jnp.float32),
        pltpu.SemaphoreType.DMA((NBUF,)),
    ],
)


def _combine_body(p_ref, o_ref):
    o_ref[...] = p_ref[0] + p_ref[1]


_combine = pl.pallas_call(
    _combine_body,
    out_shape=jax.ShapeDtypeStruct((S, D), jnp.float32),
)


def kernel(node_features, batch, ptr):
    del ptr
    batch2d = batch.reshape(N // 128, 128)
    partials = _sc_partials(node_features, batch2d)
    return _combine(partials)


# speedup vs baseline: 1.2864x; 1.0001x over previous
import jax
import jax.numpy as jnp
from jax import lax
from jax.experimental import pallas as pl
from jax.experimental.pallas import tpu as pltpu
from jax.experimental.pallas import tpu_sc as plsc

N = 320000
D = 128
S = 2048

NUM_CORES = 2
NUM_SUBCORES = 16
NUM_WORKERS = NUM_CORES * NUM_SUBCORES

CHUNK_ROWS = 128
IDX_ROWS = CHUNK_ROWS // 128
NUM_CHUNKS = N // CHUNK_ROWS
BASE_STEPS = NUM_CHUNKS // NUM_WORKERS
REMAINDER = NUM_CHUNKS - BASE_STEPS * NUM_WORKERS
MAX_STEPS = BASE_STEPS + (1 if REMAINDER else 0)
NBUF = 5
AHEAD = 3


def _sc_partials_body(nf_hbm, batch_hbm, part_hbm, rows_v, idx_v, acc, ld_sem):
    c = lax.axis_index("c")
    s = lax.axis_index("s")
    w = s * NUM_CORES + c

    n = jnp.where(w < REMAINDER, BASE_STEPS + 1, BASE_STEPS)

    def chunk_of(t):
        return jnp.where(
            t < BASE_STEPS, w + NUM_WORKERS * t, BASE_STEPS * NUM_WORKERS + w
        )

    def issue_loads(t, p):
        chunk = chunk_of(t)
        pltpu.async_copy(
            nf_hbm.at[pl.ds(chunk * CHUNK_ROWS, CHUNK_ROWS)],
            rows_v.at[p],
            ld_sem.at[p],
        )
        pltpu.async_copy(
            batch_hbm.at[pl.ds(chunk * IDX_ROWS, IDX_ROWS)],
            idx_v.at[p],
            ld_sem.at[p],
        )

    def wait_loads(p):
        pltpu.make_async_copy(
            nf_hbm.at[pl.ds(0, CHUNK_ROWS)], rows_v.at[p], ld_sem.at[p]
        ).wait()
        pltpu.make_async_copy(
            batch_hbm.at[pl.ds(0, IDX_ROWS)], idx_v.at[p], ld_sem.at[p]
        ).wait()

    def scatter(p):
        for j in range(IDX_ROWS):
            pltpu.sync_copy(
                rows_v.at[p, pl.ds(j * 128, 128)], acc.at[idx_v.at[p, j]], add=True
            )

    for t in range(AHEAD):
        issue_loads(t, t)

    def zero_row(i, _):
        for j in range(D // 16):
            rows_v[NBUF - 1, i, pl.ds(j * 16, 16)] = jnp.zeros(
                (16,), jnp.float32
            )
        return 0

    lax.fori_loop(0, 128, zero_row, 0)
    pltpu.sync_copy(
        rows_v.at[NBUF - 1, pl.ds(0, 128)], acc.at[pl.ds(s * 128, 128)]
    )
    plsc.subcore_barrier()

    def steps(i, _):
        for k in range(NBUF):
            t = NBUF * i + k

            @pl.when(t < n)
            def _():
                wait_loads(k)

                @pl.when(t + AHEAD < n)
                def _():
                    issue_loads(t + AHEAD, (k + AHEAD) % NBUF)

                scatter(k)

        return 0

    lax.fori_loop(0, (MAX_STEPS + NBUF - 1) // NBUF, steps, 0)

    plsc.subcore_barrier()
    pltpu.sync_copy(acc.at[pl.ds(s * 128, 128)], rows_v.at[0, pl.ds(0, 128)])
    pltpu.sync_copy(
        rows_v.at[0, pl.ds(0, 128)], part_hbm.at[c, pl.ds(s * 128, 128)]
    )


_sc_partials = pl.kernel(
    _sc_partials_body,
    out_type=jax.ShapeDtypeStruct((NUM_CORES, S, D), jnp.float32),
    mesh=plsc.VectorSubcoreMesh(
        core_axis_name="c", subcore_axis_name="s", num_cores=NUM_CORES
    ),
    scratch_types=[
        pltpu.VMEM((NBUF, CHUNK_ROWS, D), jnp.float32),
        pltpu.VMEM((NBUF, IDX_ROWS, 128), jnp.int32),
        pltpu.VMEM_SHARED((S, D), jnp.float32),
        pltpu.SemaphoreType.DMA((NBUF,)),
    ],
)


def _combine_body(p_ref, o_ref):
    o_ref[...] = p_ref[0] + p_ref[1]


_combine = pl.pallas_call(
    _combine_body,
    out_shape=jax.ShapeDtypeStruct((S, D), jnp.float32),
)


def kernel(node_features, batch, ptr):
    del ptr
    batch2d = batch.reshape(N // 128, 128)
    partials = _sc_partials(node_features, batch2d)
    return _combine(partials)
